# baseline (speedup 1.0000x reference)
import functools

import jax
import jax.numpy as jnp
from jax import lax
from jax.experimental import pallas as pl
from jax.experimental.pallas import tpu as pltpu
from jax.experimental.pallas import tpu_sc as plsc

U = 5000
D = 64
K = 1000
E_MSG = 320000
E_PRED = 100000
TEMP = 0.1
SSL_REG = 1e-06
PROTO_REG = 8e-08

NP = 5120
NC = 2
NS = 16
NW = NC * NS
RPT = NP // NS

CH = 128
NCHUNK = E_MSG // CH
CPW = (NCHUNK + NW - 1) // NW

CHS = 80
CHE = 160
NCHUNK_S = E_PRED // CHE
CPW_S = (NCHUNK_S + NW - 1) // NW

GPW = NP // NW


def _sc_mesh():
    return plsc.VectorSubcoreMesh(core_axis_name="c", subcore_axis_name="s",
                                  num_cores=NC, num_subcores=NS)


_SC_PARAMS = pltpu.CompilerParams(use_tc_tiling_on_sc=False,
                                  needs_layout_passes=False)

_f32 = jnp.float32


def _worker_id():
    return lax.axis_index("s") * NC + lax.axis_index("c")


def _sc_deg_gather(e0, e1, u2c_p, i2c_p, ucen, icen, ones8, zeros8):
    @functools.partial(
        pl.kernel,
        out_type=(
            jax.ShapeDtypeStruct((NC, NP, 8), _f32),
            jax.ShapeDtypeStruct((NC, NP, 8), _f32),
            jax.ShapeDtypeStruct((NP, D), _f32),
            jax.ShapeDtypeStruct((NP, D), _f32),
        ),
        mesh=_sc_mesh(),
        compiler_params=_SC_PARAMS,
        scratch_types=[
            pltpu.VMEM((4, CH), jnp.int32),
            pltpu.VMEM((CH, 8), _f32),
            pltpu.VMEM((2, CHS), jnp.int32),
            pltpu.VMEM((CHS, D), _f32),
            pltpu.VMEM_SHARED((NP, 8), _f32),
            pltpu.VMEM_SHARED((NP, 8), _f32),
            pltpu.SemaphoreType.DMA,
            pltpu.SemaphoreType.DMA,
            pltpu.SemaphoreType.DMA,
            pltpu.SemaphoreType.DMA,
        ],
    )
    def k(e0_h, e1_h, u2c_h, i2c_h, uc_h, ic_h, ones_h, z8_h,
          degu_h, degi_h, gcu_h, gci_h,
          idx_v, ones_v, cl_v, rows_v, daccu, dacci, semI0, semI1, sem, semS):
        s = lax.axis_index("s")
        c = lax.axis_index("c")
        w = _worker_id()
        semI = (semI0, semI1)

        pltpu.sync_copy(z8_h.at[pl.ds(s * RPT, RPT), :],
                        daccu.at[pl.ds(s * RPT, RPT), :])
        pltpu.sync_copy(z8_h.at[pl.ds(s * RPT, RPT), :],
                        dacci.at[pl.ds(s * RPT, RPT), :])
        pltpu.sync_copy(ones_h, ones_v)
        plsc.subcore_barrier()

        def cid_of(t):
            return w + t * NW

        def fire_idx(slot, t):
            base = cid_of(t) * CH
            pltpu.async_copy(e0_h.at[pl.ds(base, CH)],
                             idx_v.at[2 * slot], semI[slot])
            pltpu.async_copy(e1_h.at[pl.ds(base, CH)],
                             idx_v.at[2 * slot + 1], semI[slot])

        def wait_idx(slot, t):
            base = cid_of(t) * CH
            pltpu.make_async_copy(e0_h.at[pl.ds(base, CH)],
                                  idx_v.at[2 * slot], semI[slot]).wait()
            pltpu.make_async_copy(e1_h.at[pl.ds(base, CH)],
                                  idx_v.at[2 * slot + 1], semI[slot]).wait()

        fire_idx(0, 0)
        fire_idx(1, 1)

        def sub(t, slot):
            @pl.when(cid_of(t) < NCHUNK)
            def _():
                wait_idx(slot, t)
                cpu = pltpu.async_copy(ones_v, daccu.at[idx_v.at[2 * slot]],
                                       semS, add=True)
                cpi = pltpu.async_copy(ones_v, dacci.at[idx_v.at[2 * slot + 1]],
                                       semS, add=True)
                cpu.wait()
                cpi.wait()

            @pl.when(cid_of(t + 2) < NCHUNK)
            def _():
                fire_idx(slot, t + 2)

        def body(u, carry):
            sub(2 * u, 0)
            sub(2 * u + 1, 1)
            return carry

        lax.fori_loop(0, (CPW + 1) // 2, body, 0)
        plsc.subcore_barrier()
        pltpu.sync_copy(daccu.at[pl.ds(s * RPT, RPT), :],
                        degu_h.at[c, pl.ds(s * RPT, RPT), :])
        pltpu.sync_copy(dacci.at[pl.ds(s * RPT, RPT), :],
                        degi_h.at[c, pl.ds(s * RPT, RPT), :])

        gbase = w * GPW
        pltpu.sync_copy(u2c_h.at[pl.ds(gbase, CHS)], cl_v.at[0])
        pltpu.sync_copy(u2c_h.at[pl.ds(gbase + CHS, CHS)], cl_v.at[1])
        for j in range(2):
            pltpu.async_copy(uc_h.at[cl_v.at[j]], rows_v, sem).wait()
            pltpu.sync_copy(rows_v, gcu_h.at[pl.ds(gbase + j * CHS, CHS), :])
        pltpu.sync_copy(i2c_h.at[pl.ds(gbase, CHS)], cl_v.at[0])
        pltpu.sync_copy(i2c_h.at[pl.ds(gbase + CHS, CHS)], cl_v.at[1])
        for j in range(2):
            pltpu.async_copy(ic_h.at[cl_v.at[j]], rows_v, sem).wait()
            pltpu.sync_copy(rows_v, gci_h.at[pl.ds(gbase + j * CHS, CHS), :])

    return k(e0, e1, u2c_p, i2c_p, ucen, icen, ones8, zeros8)


def _sc_conv_pair(tA, tB, e0, e1, zeros64):
    @functools.partial(
        pl.kernel,
        out_type=(
            jax.ShapeDtypeStruct((NC, NP, D), _f32),
            jax.ShapeDtypeStruct((NC, NP, D), _f32),
        ),
        mesh=_sc_mesh(),
        compiler_params=_SC_PARAMS,
        scratch_types=[
            pltpu.VMEM((4, CH), jnp.int32),
            pltpu.VMEM((CH, D), _f32),
            pltpu.VMEM((CH, D), _f32),
            pltpu.VMEM((CH, D), _f32),
            pltpu.VMEM((CH, D), _f32),
            pltpu.VMEM_SHARED((NP, D), _f32),
            pltpu.VMEM_SHARED((NP, D), _f32),
            pltpu.VMEM_SHARED((U, D), _f32),
            pltpu.VMEM_SHARED((U, D), _f32),
            pltpu.SemaphoreType.DMA,
            pltpu.SemaphoreType.DMA,
            pltpu.SemaphoreType.DMA,
            pltpu.SemaphoreType.DMA,
            pltpu.SemaphoreType.DMA,
            pltpu.SemaphoreType.DMA,
        ],
    )
    def k(tA_h, tB_h, e0_h, e1_h, z_h, outB_h, outA_h,
          idx_v, rA0, rA1, rB0, rB1, accB_s, accA_s, tabA_s, tabB_s,
          semI0, semI1, semA0, semA1, semB0, semB1):
        s = lax.axis_index("s")
        c = lax.axis_index("c")
        w = _worker_id()
        rowsA = (rA0, rA1)
        rowsB = (rB0, rB1)
        semI = (semI0, semI1)
        semA = (semA0, semA1)
        semB = (semB0, semB1)

        pltpu.sync_copy(z_h.at[pl.ds(s * RPT, RPT), :],
                        accB_s.at[pl.ds(s * RPT, RPT), :])
        pltpu.sync_copy(z_h.at[pl.ds(s * RPT, RPT), :],
                        accA_s.at[pl.ds(s * RPT, RPT), :])
        @pl.when(s < NS - 1)
        def _():
            pltpu.sync_copy(tA_h.at[pl.ds(s * RPT, RPT), :],
                            tabA_s.at[pl.ds(s * RPT, RPT), :])
            pltpu.sync_copy(tB_h.at[pl.ds(s * RPT, RPT), :],
                            tabB_s.at[pl.ds(s * RPT, RPT), :])

        @pl.when(s == NS - 1)
        def _():
            tail = U - (NS - 1) * RPT
            pltpu.sync_copy(tA_h.at[pl.ds((NS - 1) * RPT, tail), :],
                            tabA_s.at[pl.ds((NS - 1) * RPT, tail), :])
            pltpu.sync_copy(tB_h.at[pl.ds((NS - 1) * RPT, tail), :],
                            tabB_s.at[pl.ds((NS - 1) * RPT, tail), :])

        plsc.subcore_barrier()

        def cid_of(t):
            return w + t * NW

        def fire_idx(slot, t):
            base = cid_of(t) * CH
            pltpu.async_copy(e0_h.at[pl.ds(base, CH)],
                             idx_v.at[2 * slot], semI[slot])
            pltpu.async_copy(e1_h.at[pl.ds(base, CH)],
                             idx_v.at[2 * slot + 1], semI[slot])

        def wait_idx(slot, t):
            base = cid_of(t) * CH
            pltpu.make_async_copy(e0_h.at[pl.ds(base, CH)],
                                  idx_v.at[2 * slot], semI[slot]).wait()
            pltpu.make_async_copy(e1_h.at[pl.ds(base, CH)],
                                  idx_v.at[2 * slot + 1], semI[slot]).wait()

        def fire_gather(slot):
            pltpu.async_copy(tabA_s.at[idx_v.at[2 * slot]],
                             rowsA[slot], semA[slot])
            pltpu.async_copy(tabB_s.at[idx_v.at[2 * slot + 1]],
                             rowsB[slot], semB[slot])

        def wait_and_scatter(slot):
            pltpu.make_async_copy(tabA_s.at[idx_v.at[2 * slot]],
                                  rowsA[slot], semA[slot]).wait()
            pltpu.sync_copy(rowsA[slot], accB_s.at[idx_v.at[2 * slot + 1]],
                            add=True)
            pltpu.make_async_copy(tabB_s.at[idx_v.at[2 * slot + 1]],
                                  rowsB[slot], semB[slot]).wait()
            pltpu.sync_copy(rowsB[slot], accA_s.at[idx_v.at[2 * slot]],
                            add=True)

        fire_idx(0, 0)
        fire_idx(1, 1)
        wait_idx(0, 0)
        fire_gather(0)

        def sub(t, slot):
            @pl.when(cid_of(t + 1) < NCHUNK)
            def _():
                wait_idx(1 - slot, t + 1)
                fire_gather(1 - slot)

            @pl.when(cid_of(t) < NCHUNK)
            def _():
                wait_and_scatter(slot)

            @pl.when(cid_of(t + 2) < NCHUNK)
            def _():
                fire_idx(slot, t + 2)

        def body(u, carry):
            sub(2 * u, 0)
            sub(2 * u + 1, 1)
            return carry

        lax.fori_loop(0, (CPW + 1) // 2, body, 0)
        plsc.subcore_barrier()
        pltpu.sync_copy(accB_s.at[pl.ds(s * RPT, RPT), :],
                        outB_h.at[c, pl.ds(s * RPT, RPT), :])
        pltpu.sync_copy(accA_s.at[pl.ds(s * RPT, RPT), :],
                        outA_h.at[c, pl.ds(s * RPT, RPT), :])

    return k(tA, tB, e0, e1, zeros64)


def _sc_scores(res_u, res_i, pu, pi, nu, ni):
    @functools.partial(
        pl.kernel,
        out_type=(
            jax.ShapeDtypeStruct((E_PRED,), _f32),
            jax.ShapeDtypeStruct((E_PRED,), _f32),
        ),
        mesh=_sc_mesh(),
        compiler_params=_SC_PARAMS,
        scratch_types=[
            pltpu.VMEM((4, CHE), jnp.int32),
            pltpu.VMEM((CHE, D), _f32),
            pltpu.VMEM((CHE, D), _f32),
            pltpu.VMEM((CHE, D), _f32),
            pltpu.VMEM((CHE, D), _f32),
            pltpu.VMEM((16, 17), _f32),
            pltpu.VMEM((CHE,), _f32),
            pltpu.SemaphoreType.DMA,
            pltpu.SemaphoreType.DMA,
            pltpu.SemaphoreType.DMA,
            pltpu.SemaphoreType.DMA,
            pltpu.SemaphoreType.DMA,
            pltpu.SemaphoreType.DMA,
        ],
    )
    def k(ru_h, ri_h, pu_h, pi_h, nu_h, ni_h, po_h, no_h,
          idx_v, rU0, rU1, rI0, rI1, tb_v, sc_v,
          semI0, semI1, semA0, semA1, semB0, semB1):
        w = _worker_id()
        rowsU = (rU0, rU1)
        rowsI = (rI0, rI1)
        semI = (semI0, semI1)
        semA = (semA0, semA1)
        semB = (semB0, semB1)

        for (a_h, b_h, out_h) in ((pu_h, pi_h, po_h), (nu_h, ni_h, no_h)):
            def cid_of(t):
                return w + t * NW

            def fire_idx(slot, t, a_h=a_h, b_h=b_h):
                base = cid_of(t) * CHE
                pltpu.async_copy(a_h.at[pl.ds(base, CHE)],
                                 idx_v.at[2 * slot], semI[slot])
                pltpu.async_copy(b_h.at[pl.ds(base, CHE)],
                                 idx_v.at[2 * slot + 1], semI[slot])

            def wait_idx(slot, t, a_h=a_h, b_h=b_h):
                base = cid_of(t) * CHE
                pltpu.make_async_copy(a_h.at[pl.ds(base, CHE)],
                                      idx_v.at[2 * slot], semI[slot]).wait()
                pltpu.make_async_copy(b_h.at[pl.ds(base, CHE)],
                                      idx_v.at[2 * slot + 1], semI[slot]).wait()

            def fire_gather(slot):
                for h in range(2):
                    pltpu.async_copy(
                        ru_h.at[idx_v.at[2 * slot, pl.ds(h * CHS, CHS)]],
                        rowsU[slot].at[pl.ds(h * CHS, CHS), :], semA[slot])
                    pltpu.async_copy(
                        ri_h.at[idx_v.at[2 * slot + 1, pl.ds(h * CHS, CHS)]],
                        rowsI[slot].at[pl.ds(h * CHS, CHS), :], semB[slot])

            def wait_gather(slot):
                for h in range(2):
                    pltpu.make_async_copy(
                        ru_h.at[idx_v.at[2 * slot, pl.ds(h * CHS, CHS)]],
                        rowsU[slot].at[pl.ds(h * CHS, CHS), :],
                        semA[slot]).wait()
                    pltpu.make_async_copy(
                        ri_h.at[idx_v.at[2 * slot + 1, pl.ds(h * CHS, CHS)]],
                        rowsI[slot].at[pl.ds(h * CHS, CHS), :],
                        semB[slot]).wait()

            def compute(slot, t, out_h=out_h):
                wait_gather(slot)
                ru_v = rowsU[slot]
                ri_v = rowsI[slot]

                def grp(g, carry):
                    for e in range(16):
                        r = g * 16 + e
                        p = (ru_v[r, pl.ds(0, 16)] * ri_v[r, pl.ds(0, 16)])
                        for q in range(1, D // 16):
                            p = p + (ru_v[r, pl.ds(q * 16, 16)]
                                     * ri_v[r, pl.ds(q * 16, 16)])
                        tb_v[e, pl.ds(0, 16)] = p
                    lids = lax.iota(jnp.int32, 16)
                    acc = plsc.load_gather(
                        tb_v, [lids, jnp.zeros((16,), jnp.int32)])
                    for cc in range(1, 16):
                        acc = acc + plsc.load_gather(
                            tb_v, [lids, jnp.full((16,), cc, jnp.int32)])
                    sc_v[pl.ds(g * 16, 16)] = acc
                    return carry

                lax.fori_loop(0, CHE // 16, grp, 0)
                pltpu.sync_copy(sc_v, out_h.at[pl.ds(cid_of(t) * CHE, CHE)])

            fire_idx(0, 0)
            fire_idx(1, 1)
            wait_idx(0, 0)
            fire_gather(0)

            def sub(t, slot):
                @pl.when(cid_of(t + 1) < NCHUNK_S)
                def _():
                    wait_idx(1 - slot, t + 1)
                    fire_gather(1 - slot)

                @pl.when(cid_of(t) < NCHUNK_S)
                def _():
                    compute(slot, t)

                @pl.when(cid_of(t + 2) < NCHUNK_S)
                def _():
                    fire_idx(slot, t + 2)

            def body(u, carry):
                sub(2 * u, 0)
                sub(2 * u + 1, 1)
                return carry

            lax.fori_loop(0, (CPW_S + 1) // 2, body, 0)

    return k(res_u, res_i, pu, pi, nu, ni)


def _tc_prep1(degu_p, degi_p, hu0, hi0):
    def body(du_r, di_r, hu_r, hi_r, dcu_r, dci_r, gu_r, gi_r,
             nu0_r, ni0_r):
        du = du_r[0] + du_r[1]
        di = di_r[0] + di_r[1]
        dcu = lax.rsqrt(jnp.maximum(du, 1.0))
        dci = lax.rsqrt(jnp.maximum(di, 1.0))
        dcu_r[...] = dcu
        dci_r[...] = dci
        hu0v = hu_r[...]
        hi0v = hi_r[...]
        gu_r[...] = hu0v * dcu[:U, :1]
        gi_r[...] = hi0v * dci[:U, :1]
        nu0_r[...] = _row_normalize(hu0v)
        ni0_r[...] = _row_normalize(hi0v)

    return pl.pallas_call(
        body,
        out_shape=(
            jax.ShapeDtypeStruct((NP, 8), _f32),
            jax.ShapeDtypeStruct((NP, 8), _f32),
            jax.ShapeDtypeStruct((U, D), _f32),
            jax.ShapeDtypeStruct((U, D), _f32),
            jax.ShapeDtypeStruct((U, D), _f32),
            jax.ShapeDtypeStruct((U, D), _f32),
        ),
    )(degu_p, degi_p, hu0, hi0)


def _tc_prep2(si1_p, su1_p, dcu, dci):
    def body(si_r, su_r, dcu_r, dci_r, t3_r, t4_r):
        si = (si_r[0] + si_r[1])[:U]
        su = (su_r[0] + su_r[1])[:U]
        dci2 = dci_r[:U, :1] * dci_r[:U, :1]
        dcu2 = dcu_r[:U, :1] * dcu_r[:U, :1]
        t3_r[...] = dci2 * si
        t4_r[...] = dcu2 * su

    return pl.pallas_call(
        body,
        out_shape=(
            jax.ShapeDtypeStruct((U, D), _f32),
            jax.ShapeDtypeStruct((U, D), _f32),
        ),
    )(si1_p, su1_p, dcu, dci)


def _row_normalize(x):
    ss = jnp.sum(x * x, axis=1, keepdims=True)
    return x / jnp.maximum(jnp.sqrt(ss), 1e-12)


def _tc_assemble(hu0, hi0, su1_p, su2_p, si1_p, si2_p, dcu, dci):
    def body(hu_r, hi_r, su1_r, su2_r, si1_r, si2_r, dcu_r, dci_r,
             resu_r, resi_r, nu2_r, ni2_r):
        su2 = (su2_r[0] + su2_r[1])[:U]
        si2 = (si2_r[0] + si2_r[1])[:U]
        su_all = (su1_r[0] + su1_r[1])[:U] + su2
        si_all = (si1_r[0] + si1_r[1])[:U] + si2
        resu_r[...] = (hu_r[...] + dcu_r[:U, :1] * su_all) * (1.0 / 3.0)
        resi_r[...] = (hi_r[...] + dci_r[:U, :1] * si_all) * (1.0 / 3.0)
        nu2_r[...] = _row_normalize(su2)
        ni2_r[...] = _row_normalize(si2)

    return pl.pallas_call(
        body,
        out_shape=tuple(jax.ShapeDtypeStruct((U, D), _f32) for _ in range(4)),
    )(hu0, hi0, su1_p, su2_p, si1_p, si2_p, dcu, dci)


RB = 1000
NRB = U // RB

_CONTRACT = (((1,), (1,)), ((), ()))


def _lse(a, b):
    m = lax.dot_general(a.astype(jnp.bfloat16), b.astype(jnp.bfloat16),
                        _CONTRACT, preferred_element_type=_f32)
    return jnp.sum(jnp.log(jnp.sum(jnp.exp(m * (1.0 / TEMP)), axis=1)))


def _tc_proto_loss(nu0, ni0, ucen, icen, gcu, gci):
    def body(nu0b_r, ni0b_r, uc_r, ic_r, gcu_r, gci_r, proto_r):
        i = pl.program_id(0)

        @pl.when(i == 0)
        def _():
            proto_r[...] = jnp.zeros_like(proto_r)

        b_u = nu0b_r[...]
        b_i = ni0b_r[...]
        pr_u = _lse(b_u, uc_r[...]) - jnp.sum(b_u * gcu_r[...]) * (1.0 / TEMP)
        pr_i = _lse(b_i, ic_r[...]) - jnp.sum(b_i * gci_r[...]) * (1.0 / TEMP)
        proto_r[...] += PROTO_REG * (pr_u + pr_i)

    blk = pl.BlockSpec((RB, D), lambda i: (i, 0))
    fullk = pl.BlockSpec((K, D), lambda i: (0, 0))
    out_blk = pl.BlockSpec((1, 1), lambda i: (0, 0))
    return pl.pallas_call(
        body,
        grid=(NRB,),
        in_specs=[blk, blk, fullk, fullk, blk, blk],
        out_specs=out_blk,
        out_shape=jax.ShapeDtypeStruct((1, 1), _f32),
    )(nu0, ni0, ucen, icen, gcu, gci)


def _tc_ssl_loss(nu2, nu0, ni2, ni0):
    def body(nu2_r, nu0f_r, nu0b_r, ni2_r, ni0f_r, ni0b_r, ssl_r):
        i = pl.program_id(0)

        @pl.when(i == 0)
        def _():
            ssl_r[...] = jnp.zeros_like(ssl_r)

        a_u = nu2_r[...]
        a_i = ni2_r[...]
        ssl_u = _lse(a_u, nu0f_r[...]) - jnp.sum(a_u * nu0b_r[...]) / TEMP
        ssl_i = _lse(a_i, ni0f_r[...]) - jnp.sum(a_i * ni0b_r[...]) / TEMP
        ssl_r[...] += SSL_REG * (ssl_u + ssl_i)

    blk = pl.BlockSpec((RB, D), lambda i: (i, 0))
    full = pl.BlockSpec((U, D), lambda i: (0, 0))
    out_blk = pl.BlockSpec((1, 1), lambda i: (0, 0))
    return pl.pallas_call(
        body,
        grid=(NRB,),
        in_specs=[blk, full, blk, blk, full, blk],
        out_specs=out_blk,
        out_shape=jax.ShapeDtypeStruct((1, 1), _f32),
    )(nu2, nu0, nu0, ni2, ni0, ni0)


def kernel(user_emb, item_emb, user_centroids, item_centroids,
           user_2cluster, item_2cluster, msg_edges, pos_edges, neg_edges):
    e0 = msg_edges[0]
    e1 = msg_edges[1]
    u2c_p = jnp.pad(user_2cluster.astype(jnp.int32), (0, NP - U))
    i2c_p = jnp.pad(item_2cluster.astype(jnp.int32), (0, NP - U))
    ones8 = jnp.ones((CH, 8), _f32)
    zeros8 = jnp.zeros((NP, 8), _f32)
    zeros64 = jnp.zeros((NP, D), _f32)

    degu_p, degi_p, gcu, gci = _sc_deg_gather(
        e0, e1, u2c_p, i2c_p, user_centroids, item_centroids, ones8, zeros8)

    dcu, dci, gu0, gi0, nu0, ni0 = _tc_prep1(degu_p, degi_p,
                                             user_emb, item_emb)

    proto2d = _tc_proto_loss(nu0, ni0, user_centroids, item_centroids,
                             gcu[:U], gci[:U])

    si1_p, su1_p = _sc_conv_pair(gu0, gi0, e0, e1, zeros64)

    t3, t4 = _tc_prep2(si1_p, su1_p, dcu, dci)

    si2_p, su2_p = _sc_conv_pair(t4, t3, e0, e1, zeros64)

    res_u, res_i, nu2, ni2 = _tc_assemble(
        user_emb, item_emb, su1_p, su2_p, si1_p, si2_p, dcu, dci)

    ssl2d = _tc_ssl_loss(nu2, nu0, ni2, ni0)

    pos_s, neg_s = _sc_scores(res_u, res_i,
                              pos_edges[0], pos_edges[1],
                              neg_edges[0], neg_edges[1])

    return (pos_s[:, None], neg_s[:, None], ssl2d[0, 0], proto2d[0, 0])


# speedup vs baseline: 1.2536x; 1.2536x over previous
import functools

import jax
import jax.numpy as jnp
from jax import lax
from jax.experimental import pallas as pl
from jax.experimental.pallas import tpu as pltpu
from jax.experimental.pallas import tpu_sc as plsc

U = 5000
D = 64
K = 1000
E_MSG = 320000
E_PRED = 100000
TEMP = 0.1
SSL_REG = 1e-06
PROTO_REG = 8e-08

NP = 5120
NC = 2
NS = 16
NW = NC * NS
RPT = NP // NS

CH = 128
NCHUNK = E_MSG // CH
CPW = (NCHUNK + NW - 1) // NW

CH2 = 256
NCHUNK2 = E_MSG // CH2
CPW2 = (NCHUNK2 + NW - 1) // NW

CHS = 80
CHE = 160
NCHUNK_S = E_PRED // CHE
CPW_S = (NCHUNK_S + NW - 1) // NW

GPW = NP // NW


def _sc_mesh():
    return plsc.VectorSubcoreMesh(core_axis_name="c", subcore_axis_name="s",
                                  num_cores=NC, num_subcores=NS)


_SC_PARAMS = pltpu.CompilerParams(use_tc_tiling_on_sc=False,
                                  needs_layout_passes=False)

_f32 = jnp.float32


def _worker_id():
    return lax.axis_index("s") * NC + lax.axis_index("c")


def _sc_deg_gather(e0, e1, u2c_p, i2c_p, ucen, icen, ones8, zeros8):
    @functools.partial(
        pl.kernel,
        out_type=(
            jax.ShapeDtypeStruct((NC, NP, 8), _f32),
            jax.ShapeDtypeStruct((NC, NP, 8), _f32),
            jax.ShapeDtypeStruct((NP, D), _f32),
            jax.ShapeDtypeStruct((NP, D), _f32),
        ),
        mesh=_sc_mesh(),
        compiler_params=_SC_PARAMS,
        scratch_types=[
            pltpu.VMEM((4, CH), jnp.int32),
            pltpu.VMEM((CH, 8), _f32),
            pltpu.VMEM((2, CHS), jnp.int32),
            pltpu.VMEM((CHS, D), _f32),
            pltpu.VMEM_SHARED((NP, 8), _f32),
            pltpu.VMEM_SHARED((NP, 8), _f32),
            pltpu.SemaphoreType.DMA,
            pltpu.SemaphoreType.DMA,
            pltpu.SemaphoreType.DMA,
            pltpu.SemaphoreType.DMA,
        ],
    )
    def k(e0_h, e1_h, u2c_h, i2c_h, uc_h, ic_h, ones_h, z8_h,
          degu_h, degi_h, gcu_h, gci_h,
          idx_v, ones_v, cl_v, rows_v, daccu, dacci, semI0, semI1, sem, semS):
        s = lax.axis_index("s")
        c = lax.axis_index("c")
        w = _worker_id()
        semI = (semI0, semI1)

        pltpu.sync_copy(z8_h.at[pl.ds(s * RPT, RPT), :],
                        daccu.at[pl.ds(s * RPT, RPT), :])
        pltpu.sync_copy(z8_h.at[pl.ds(s * RPT, RPT), :],
                        dacci.at[pl.ds(s * RPT, RPT), :])
        pltpu.sync_copy(ones_h, ones_v)
        plsc.subcore_barrier()

        def cid_of(t):
            return w + t * NW

        def fire_idx(slot, t):
            base = cid_of(t) * CH
            pltpu.async_copy(e0_h.at[pl.ds(base, CH)],
                             idx_v.at[2 * slot], semI[slot])
            pltpu.async_copy(e1_h.at[pl.ds(base, CH)],
                             idx_v.at[2 * slot + 1], semI[slot])

        def wait_idx(slot, t):
            base = cid_of(t) * CH
            pltpu.make_async_copy(e0_h.at[pl.ds(base, CH)],
                                  idx_v.at[2 * slot], semI[slot]).wait()
            pltpu.make_async_copy(e1_h.at[pl.ds(base, CH)],
                                  idx_v.at[2 * slot + 1], semI[slot]).wait()

        fire_idx(0, 0)
        fire_idx(1, 1)

        def sub(t, slot):
            @pl.when(cid_of(t) < NCHUNK)
            def _():
                wait_idx(slot, t)
                cpu = pltpu.async_copy(ones_v, daccu.at[idx_v.at[2 * slot]],
                                       semS, add=True)
                cpi = pltpu.async_copy(ones_v, dacci.at[idx_v.at[2 * slot + 1]],
                                       semS, add=True)
                cpu.wait()
                cpi.wait()

            @pl.when(cid_of(t + 2) < NCHUNK)
            def _():
                fire_idx(slot, t + 2)

        def body(u, carry):
            sub(2 * u, 0)
            sub(2 * u + 1, 1)
            return carry

        lax.fori_loop(0, (CPW + 1) // 2, body, 0)
        plsc.subcore_barrier()
        pltpu.sync_copy(daccu.at[pl.ds(s * RPT, RPT), :],
                        degu_h.at[c, pl.ds(s * RPT, RPT), :])
        pltpu.sync_copy(dacci.at[pl.ds(s * RPT, RPT), :],
                        degi_h.at[c, pl.ds(s * RPT, RPT), :])

        gbase = w * GPW
        pltpu.sync_copy(u2c_h.at[pl.ds(gbase, CHS)], cl_v.at[0])
        pltpu.sync_copy(u2c_h.at[pl.ds(gbase + CHS, CHS)], cl_v.at[1])
        for j in range(2):
            pltpu.async_copy(uc_h.at[cl_v.at[j]], rows_v, sem).wait()
            pltpu.sync_copy(rows_v, gcu_h.at[pl.ds(gbase + j * CHS, CHS), :])
        pltpu.sync_copy(i2c_h.at[pl.ds(gbase, CHS)], cl_v.at[0])
        pltpu.sync_copy(i2c_h.at[pl.ds(gbase + CHS, CHS)], cl_v.at[1])
        for j in range(2):
            pltpu.async_copy(ic_h.at[cl_v.at[j]], rows_v, sem).wait()
            pltpu.sync_copy(rows_v, gci_h.at[pl.ds(gbase + j * CHS, CHS), :])

    return k(e0, e1, u2c_p, i2c_p, ucen, icen, ones8, zeros8)


def _sc_conv_pair(tA, tB, e0, e1, zeros64):
    @functools.partial(
        pl.kernel,
        out_type=(
            jax.ShapeDtypeStruct((NC, NP, D), _f32),
            jax.ShapeDtypeStruct((NC, NP, D), _f32),
        ),
        mesh=_sc_mesh(),
        compiler_params=_SC_PARAMS,
        scratch_types=[
            pltpu.VMEM((8, CH2), jnp.int32),
            pltpu.VMEM((CH2, D), _f32),
            pltpu.VMEM((CH2, D), _f32),
            pltpu.VMEM((CH2, D), _f32),
            pltpu.VMEM((CH2, D), _f32),
            pltpu.VMEM_SHARED((NP, D), _f32),
            pltpu.VMEM_SHARED((NP, D), _f32),
        ] + [pltpu.SemaphoreType.DMA] * 10,
    )
    def k(tA_h, tB_h, e0_h, e1_h, z_h, outB_h, outA_h,
          idx_v, rA0, rA1, rB0, rB1, accB_s, accA_s,
          semI0, semI1, semI2, semI3, semA0, semA1, semB0, semB1,
          semS0, semS1):
        s = lax.axis_index("s")
        c = lax.axis_index("c")
        w = _worker_id()
        rowsA = (rA0, rA1)
        rowsB = (rB0, rB1)
        semI = (semI0, semI1, semI2, semI3)
        semA = (semA0, semA1)
        semB = (semB0, semB1)
        semS = (semS0, semS1)

        pltpu.sync_copy(z_h.at[pl.ds(s * RPT, RPT), :],
                        accB_s.at[pl.ds(s * RPT, RPT), :])
        pltpu.sync_copy(z_h.at[pl.ds(s * RPT, RPT), :],
                        accA_s.at[pl.ds(s * RPT, RPT), :])
        plsc.subcore_barrier()

        def cid_of(t):
            return w + t * NW

        def fire_idx(s4, t):
            base = cid_of(t) * CH2
            pltpu.async_copy(e0_h.at[pl.ds(base, CH2)],
                             idx_v.at[2 * s4], semI[s4])
            pltpu.async_copy(e1_h.at[pl.ds(base, CH2)],
                             idx_v.at[2 * s4 + 1], semI[s4])

        def wait_idx(s4, t):
            base = cid_of(t) * CH2
            pltpu.make_async_copy(e0_h.at[pl.ds(base, CH2)],
                                  idx_v.at[2 * s4], semI[s4]).wait()
            pltpu.make_async_copy(e1_h.at[pl.ds(base, CH2)],
                                  idx_v.at[2 * s4 + 1], semI[s4]).wait()

        def _gather_args(s2, s4):
            for h in range(2):
                yield (tA_h.at[idx_v.at[2 * s4, pl.ds(h * CH, CH)]],
                       rowsA[s2].at[pl.ds(h * CH, CH), :], semA[s2])
                yield (tB_h.at[idx_v.at[2 * s4 + 1, pl.ds(h * CH, CH)]],
                       rowsB[s2].at[pl.ds(h * CH, CH), :], semB[s2])

        def fire_gather(s2, s4):
            for src, dst, sem in _gather_args(s2, s4):
                pltpu.async_copy(src, dst, sem)

        def wait_gather(s2, s4):
            for src, dst, sem in _gather_args(s2, s4):
                pltpu.make_async_copy(src, dst, sem).wait()

        def _scatter_args(s2, s4):
            for h in range(2):
                yield (rowsA[s2].at[pl.ds(h * CH, CH), :],
                       accB_s.at[idx_v.at[2 * s4 + 1, pl.ds(h * CH, CH)]])
                yield (rowsB[s2].at[pl.ds(h * CH, CH), :],
                       accA_s.at[idx_v.at[2 * s4, pl.ds(h * CH, CH)]])

        def fire_scatter(s2, s4):
            for src, dst in _scatter_args(s2, s4):
                pltpu.async_copy(src, dst, semS[s2], add=True)

        def wait_scatter(s2, s4):
            for src, dst in _scatter_args(s2, s4):
                pltpu.make_async_copy(src, dst, semS[s2]).wait()

        fire_idx(0, 0)
        fire_idx(1, 1)
        wait_idx(0, 0)
        fire_gather(0, 0)

        def sub(u, j):
            t = 4 * u + j
            s2 = j % 2
            prev_ok = cid_of(t - 1) < NCHUNK2
            if j == 0:
                prev_ok = jnp.logical_and(u >= 1, prev_ok)

            @pl.when(prev_ok)
            def _():
                wait_scatter(1 - s2, (j - 1) % 4)

            @pl.when(cid_of(t + 1) < NCHUNK2)
            def _():
                wait_idx((j + 1) % 4, t + 1)
                fire_gather(1 - s2, (j + 1) % 4)

            @pl.when(cid_of(t) < NCHUNK2)
            def _():
                wait_gather(s2, j)
                fire_scatter(s2, j)

            @pl.when(cid_of(t + 2) < NCHUNK2)
            def _():
                fire_idx((j + 2) % 4, t + 2)

        def body(u, carry):
            for j in range(4):
                sub(u, j)
            return carry

        lax.fori_loop(0, CPW2 // 4, body, 0)

        @pl.when(cid_of(CPW2 - 1) < NCHUNK2)
        def _():
            wait_scatter((CPW2 - 1) % 2, (CPW2 - 1) % 4)

        plsc.subcore_barrier()
        pltpu.sync_copy(accB_s.at[pl.ds(s * RPT, RPT), :],
                        outB_h.at[c, pl.ds(s * RPT, RPT), :])
        pltpu.sync_copy(accA_s.at[pl.ds(s * RPT, RPT), :],
                        outA_h.at[c, pl.ds(s * RPT, RPT), :])

    return k(tA, tB, e0, e1, zeros64)


def _sc_scores(res_u, res_i, pu, pi, nu, ni):
    @functools.partial(
        pl.kernel,
        out_type=(
            jax.ShapeDtypeStruct((E_PRED,), _f32),
            jax.ShapeDtypeStruct((E_PRED,), _f32),
        ),
        mesh=_sc_mesh(),
        compiler_params=_SC_PARAMS,
        scratch_types=[
            pltpu.VMEM((4, CHE), jnp.int32),
            pltpu.VMEM((CHE, D), _f32),
            pltpu.VMEM((CHE, D), _f32),
            pltpu.VMEM((CHE, D), _f32),
            pltpu.VMEM((CHE, D), _f32),
            pltpu.VMEM((16, 17), _f32),
            pltpu.VMEM((CHE,), _f32),
            pltpu.SemaphoreType.DMA,
            pltpu.SemaphoreType.DMA,
            pltpu.SemaphoreType.DMA,
            pltpu.SemaphoreType.DMA,
            pltpu.SemaphoreType.DMA,
            pltpu.SemaphoreType.DMA,
        ],
    )
    def k(ru_h, ri_h, pu_h, pi_h, nu_h, ni_h, po_h, no_h,
          idx_v, rU0, rU1, rI0, rI1, tb_v, sc_v,
          semI0, semI1, semA0, semA1, semB0, semB1):
        w = _worker_id()
        rowsU = (rU0, rU1)
        rowsI = (rI0, rI1)
        semI = (semI0, semI1)
        semA = (semA0, semA1)
        semB = (semB0, semB1)

        for (a_h, b_h, out_h) in ((pu_h, pi_h, po_h), (nu_h, ni_h, no_h)):
            def cid_of(t):
                return w + t * NW

            def fire_idx(slot, t, a_h=a_h, b_h=b_h):
                base = cid_of(t) * CHE
                pltpu.async_copy(a_h.at[pl.ds(base, CHE)],
                                 idx_v.at[2 * slot], semI[slot])
                pltpu.async_copy(b_h.at[pl.ds(base, CHE)],
                                 idx_v.at[2 * slot + 1], semI[slot])

            def wait_idx(slot, t, a_h=a_h, b_h=b_h):
                base = cid_of(t) * CHE
                pltpu.make_async_copy(a_h.at[pl.ds(base, CHE)],
                                      idx_v.at[2 * slot], semI[slot]).wait()
                pltpu.make_async_copy(b_h.at[pl.ds(base, CHE)],
                                      idx_v.at[2 * slot + 1], semI[slot]).wait()

            def fire_gather(slot):
                for h in range(2):
                    pltpu.async_copy(
                        ru_h.at[idx_v.at[2 * slot, pl.ds(h * CHS, CHS)]],
                        rowsU[slot].at[pl.ds(h * CHS, CHS), :], semA[slot])
                    pltpu.async_copy(
                        ri_h.at[idx_v.at[2 * slot + 1, pl.ds(h * CHS, CHS)]],
                        rowsI[slot].at[pl.ds(h * CHS, CHS), :], semB[slot])

            def wait_gather(slot):
                for h in range(2):
                    pltpu.make_async_copy(
                        ru_h.at[idx_v.at[2 * slot, pl.ds(h * CHS, CHS)]],
                        rowsU[slot].at[pl.ds(h * CHS, CHS), :],
                        semA[slot]).wait()
                    pltpu.make_async_copy(
                        ri_h.at[idx_v.at[2 * slot + 1, pl.ds(h * CHS, CHS)]],
                        rowsI[slot].at[pl.ds(h * CHS, CHS), :],
                        semB[slot]).wait()

            def compute(slot, t, out_h=out_h):
                wait_gather(slot)
                ru_v = rowsU[slot]
                ri_v = rowsI[slot]

                def grp(g, carry):
                    for e in range(16):
                        r = g * 16 + e
                        p = (ru_v[r, pl.ds(0, 16)] * ri_v[r, pl.ds(0, 16)])
                        for q in range(1, D // 16):
                            p = p + (ru_v[r, pl.ds(q * 16, 16)]
                                     * ri_v[r, pl.ds(q * 16, 16)])
                        tb_v[e, pl.ds(0, 16)] = p
                    lids = lax.iota(jnp.int32, 16)
                    acc = plsc.load_gather(
                        tb_v, [lids, jnp.zeros((16,), jnp.int32)])
                    for cc in range(1, 16):
                        acc = acc + plsc.load_gather(
                            tb_v, [lids, jnp.full((16,), cc, jnp.int32)])
                    sc_v[pl.ds(g * 16, 16)] = acc
                    return carry

                lax.fori_loop(0, CHE // 16, grp, 0)
                pltpu.sync_copy(sc_v, out_h.at[pl.ds(cid_of(t) * CHE, CHE)])

            fire_idx(0, 0)
            fire_idx(1, 1)
            wait_idx(0, 0)
            fire_gather(0)

            def sub(t, slot):
                @pl.when(cid_of(t + 1) < NCHUNK_S)
                def _():
                    wait_idx(1 - slot, t + 1)
                    fire_gather(1 - slot)

                @pl.when(cid_of(t) < NCHUNK_S)
                def _():
                    compute(slot, t)

                @pl.when(cid_of(t + 2) < NCHUNK_S)
                def _():
                    fire_idx(slot, t + 2)

            def body(u, carry):
                sub(2 * u, 0)
                sub(2 * u + 1, 1)
                return carry

            lax.fori_loop(0, (CPW_S + 1) // 2, body, 0)

    return k(res_u, res_i, pu, pi, nu, ni)


def _tc_prep1(degu_p, degi_p, hu0, hi0):
    def body(du_r, di_r, hu_r, hi_r, dcu_r, dci_r, gu_r, gi_r,
             nu0_r, ni0_r):
        du = du_r[0] + du_r[1]
        di = di_r[0] + di_r[1]
        dcu = lax.rsqrt(jnp.maximum(du, 1.0))
        dci = lax.rsqrt(jnp.maximum(di, 1.0))
        dcu_r[...] = dcu
        dci_r[...] = dci
        hu0v = hu_r[...]
        hi0v = hi_r[...]
        gu_r[...] = hu0v * dcu[:U, :1]
        gi_r[...] = hi0v * dci[:U, :1]
        nu0_r[...] = _row_normalize(hu0v)
        ni0_r[...] = _row_normalize(hi0v)

    return pl.pallas_call(
        body,
        out_shape=(
            jax.ShapeDtypeStruct((NP, 8), _f32),
            jax.ShapeDtypeStruct((NP, 8), _f32),
            jax.ShapeDtypeStruct((U, D), _f32),
            jax.ShapeDtypeStruct((U, D), _f32),
            jax.ShapeDtypeStruct((U, D), _f32),
            jax.ShapeDtypeStruct((U, D), _f32),
        ),
    )(degu_p, degi_p, hu0, hi0)


def _tc_prep2(si1_p, su1_p, dcu, dci):
    def body(si_r, su_r, dcu_r, dci_r, t3_r, t4_r):
        si = (si_r[0] + si_r[1])[:U]
        su = (su_r[0] + su_r[1])[:U]
        dci2 = dci_r[:U, :1] * dci_r[:U, :1]
        dcu2 = dcu_r[:U, :1] * dcu_r[:U, :1]
        t3_r[...] = dci2 * si
        t4_r[...] = dcu2 * su

    return pl.pallas_call(
        body,
        out_shape=(
            jax.ShapeDtypeStruct((U, D), _f32),
            jax.ShapeDtypeStruct((U, D), _f32),
        ),
    )(si1_p, su1_p, dcu, dci)


def _row_normalize(x):
    ss = jnp.sum(x * x, axis=1, keepdims=True)
    return x / jnp.maximum(jnp.sqrt(ss), 1e-12)


def _tc_assemble(hu0, hi0, su1_p, su2_p, si1_p, si2_p, dcu, dci):
    def body(hu_r, hi_r, su1_r, su2_r, si1_r, si2_r, dcu_r, dci_r,
             resu_r, resi_r, nu2_r, ni2_r):
        su2 = (su2_r[0] + su2_r[1])[:U]
        si2 = (si2_r[0] + si2_r[1])[:U]
        su_all = (su1_r[0] + su1_r[1])[:U] + su2
        si_all = (si1_r[0] + si1_r[1])[:U] + si2
        resu_r[...] = (hu_r[...] + dcu_r[:U, :1] * su_all) * (1.0 / 3.0)
        resi_r[...] = (hi_r[...] + dci_r[:U, :1] * si_all) * (1.0 / 3.0)
        nu2_r[...] = _row_normalize(su2)
        ni2_r[...] = _row_normalize(si2)

    return pl.pallas_call(
        body,
        out_shape=tuple(jax.ShapeDtypeStruct((U, D), _f32) for _ in range(4)),
    )(hu0, hi0, su1_p, su2_p, si1_p, si2_p, dcu, dci)


RB = 1000
NRB = U // RB

_CONTRACT = (((1,), (1,)), ((), ()))


def _lse(a, b):
    m = lax.dot_general(a.astype(jnp.bfloat16), b.astype(jnp.bfloat16),
                        _CONTRACT, preferred_element_type=_f32)
    return jnp.sum(jnp.log(jnp.sum(jnp.exp(m * (1.0 / TEMP)), axis=1)))


def _tc_proto_loss(nu0, ni0, ucen, icen, gcu, gci):
    def body(nu0b_r, ni0b_r, uc_r, ic_r, gcu_r, gci_r, proto_r):
        i = pl.program_id(0)

        @pl.when(i == 0)
        def _():
            proto_r[...] = jnp.zeros_like(proto_r)

        b_u = nu0b_r[...]
        b_i = ni0b_r[...]
        pr_u = _lse(b_u, uc_r[...]) - jnp.sum(b_u * gcu_r[...]) * (1.0 / TEMP)
        pr_i = _lse(b_i, ic_r[...]) - jnp.sum(b_i * gci_r[...]) * (1.0 / TEMP)
        proto_r[...] += PROTO_REG * (pr_u + pr_i)

    blk = pl.BlockSpec((RB, D), lambda i: (i, 0))
    fullk = pl.BlockSpec((K, D), lambda i: (0, 0))
    out_blk = pl.BlockSpec((1, 1), lambda i: (0, 0))
    return pl.pallas_call(
        body,
        grid=(NRB,),
        in_specs=[blk, blk, fullk, fullk, blk, blk],
        out_specs=out_blk,
        out_shape=jax.ShapeDtypeStruct((1, 1), _f32),
    )(nu0, ni0, ucen, icen, gcu, gci)


def _tc_ssl_loss(nu2, nu0, ni2, ni0):
    def body(nu2_r, nu0f_r, nu0b_r, ni2_r, ni0f_r, ni0b_r, ssl_r):
        i = pl.program_id(0)

        @pl.when(i == 0)
        def _():
            ssl_r[...] = jnp.zeros_like(ssl_r)

        a_u = nu2_r[...]
        a_i = ni2_r[...]
        ssl_u = _lse(a_u, nu0f_r[...]) - jnp.sum(a_u * nu0b_r[...]) / TEMP
        ssl_i = _lse(a_i, ni0f_r[...]) - jnp.sum(a_i * ni0b_r[...]) / TEMP
        ssl_r[...] += SSL_REG * (ssl_u + ssl_i)

    blk = pl.BlockSpec((RB, D), lambda i: (i, 0))
    full = pl.BlockSpec((U, D), lambda i: (0, 0))
    out_blk = pl.BlockSpec((1, 1), lambda i: (0, 0))
    return pl.pallas_call(
        body,
        grid=(NRB,),
        in_specs=[blk, full, blk, blk, full, blk],
        out_specs=out_blk,
        out_shape=jax.ShapeDtypeStruct((1, 1), _f32),
    )(nu2, nu0, nu0, ni2, ni0, ni0)


def kernel(user_emb, item_emb, user_centroids, item_centroids,
           user_2cluster, item_2cluster, msg_edges, pos_edges, neg_edges):
    e0 = msg_edges[0]
    e1 = msg_edges[1]
    u2c_p = jnp.pad(user_2cluster.astype(jnp.int32), (0, NP - U))
    i2c_p = jnp.pad(item_2cluster.astype(jnp.int32), (0, NP - U))
    ones8 = jnp.ones((CH, 8), _f32)
    zeros8 = jnp.zeros((NP, 8), _f32)
    zeros64 = jnp.zeros((NP, D), _f32)

    degu_p, degi_p, gcu, gci = _sc_deg_gather(
        e0, e1, u2c_p, i2c_p, user_centroids, item_centroids, ones8, zeros8)

    dcu, dci, gu0, gi0, nu0, ni0 = _tc_prep1(degu_p, degi_p,
                                             user_emb, item_emb)

    proto2d = _tc_proto_loss(nu0, ni0, user_centroids, item_centroids,
                             gcu[:U], gci[:U])

    si1_p, su1_p = _sc_conv_pair(gu0, gi0, e0, e1, zeros64)

    t3, t4 = _tc_prep2(si1_p, su1_p, dcu, dci)

    si2_p, su2_p = _sc_conv_pair(t4, t3, e0, e1, zeros64)

    res_u, res_i, nu2, ni2 = _tc_assemble(
        user_emb, item_emb, su1_p, su2_p, si1_p, si2_p, dcu, dci)

    ssl2d = _tc_ssl_loss(nu2, nu0, ni2, ni0)

    pos_s, neg_s = _sc_scores(res_u, res_i,
                              pos_edges[0], pos_edges[1],
                              neg_edges[0], neg_edges[1])

    return (pos_s[:, None], neg_s[:, None], ssl2d[0, 0], proto2d[0, 0])


# speedup vs baseline: 1.2695x; 1.0127x over previous
import functools

import jax
import jax.numpy as jnp
from jax import lax
from jax.experimental import pallas as pl
from jax.experimental.pallas import tpu as pltpu
from jax.experimental.pallas import tpu_sc as plsc

U = 5000
D = 64
K = 1000
E_MSG = 320000
E_PRED = 100000
TEMP = 0.1
SSL_REG = 1e-06
PROTO_REG = 8e-08

NP = 5120
NC = 2
NS = 16
NW = NC * NS
RPT = NP // NS

CH = 128
NCHUNK = E_MSG // CH
CPW = (NCHUNK + NW - 1) // NW

CH2 = 256
NCHUNK2 = E_MSG // CH2
CPW2 = (NCHUNK2 + NW - 1) // NW

CHS = 80
CHE = 160
NCHUNK_S = E_PRED // CHE
CPW_S = (NCHUNK_S + NW - 1) // NW

GPW = NP // NW


def _sc_mesh():
    return plsc.VectorSubcoreMesh(core_axis_name="c", subcore_axis_name="s",
                                  num_cores=NC, num_subcores=NS)


_SC_PARAMS = pltpu.CompilerParams(use_tc_tiling_on_sc=False,
                                  needs_layout_passes=False)

_f32 = jnp.float32


def _worker_id():
    return lax.axis_index("s") * NC + lax.axis_index("c")


def _sc_deg_gather(e0, e1, u2c_p, i2c_p, ucen, icen, ones8, zeros8):
    @functools.partial(
        pl.kernel,
        out_type=(
            jax.ShapeDtypeStruct((NC, NP, 8), _f32),
            jax.ShapeDtypeStruct((NC, NP, 8), _f32),
            jax.ShapeDtypeStruct((NP, D), _f32),
            jax.ShapeDtypeStruct((NP, D), _f32),
        ),
        mesh=_sc_mesh(),
        compiler_params=_SC_PARAMS,
        scratch_types=[
            pltpu.VMEM((8, CH), jnp.int32),
            pltpu.VMEM((CH, 8), _f32),
            pltpu.VMEM((2, CHS), jnp.int32),
            pltpu.VMEM((CHS, D), _f32),
            pltpu.VMEM_SHARED((NP, 8), _f32),
            pltpu.VMEM_SHARED((NP, 8), _f32),
        ] + [pltpu.SemaphoreType.DMA] * 9,
    )
    def k(e0_h, e1_h, u2c_h, i2c_h, uc_h, ic_h, ones_h, z8_h,
          degu_h, degi_h, gcu_h, gci_h,
          idx_v, ones_v, cl_v, rows_v, daccu, dacci,
          semI0, semI1, semI2, semI3, sem, semS0, semS1, semS2, semS3):
        s = lax.axis_index("s")
        c = lax.axis_index("c")
        w = _worker_id()
        semI = (semI0, semI1, semI2, semI3)
        semS = (semS0, semS1, semS2, semS3)

        pltpu.sync_copy(z8_h.at[pl.ds(s * RPT, RPT), :],
                        daccu.at[pl.ds(s * RPT, RPT), :])
        pltpu.sync_copy(z8_h.at[pl.ds(s * RPT, RPT), :],
                        dacci.at[pl.ds(s * RPT, RPT), :])
        pltpu.sync_copy(ones_h, ones_v)
        plsc.subcore_barrier()

        def cid_of(t):
            return w + t * NW

        def fire_idx(slot, t):
            base = cid_of(t) * CH
            pltpu.async_copy(e0_h.at[pl.ds(base, CH)],
                             idx_v.at[2 * slot], semI[slot])
            pltpu.async_copy(e1_h.at[pl.ds(base, CH)],
                             idx_v.at[2 * slot + 1], semI[slot])

        def wait_idx(slot, t):
            base = cid_of(t) * CH
            pltpu.make_async_copy(e0_h.at[pl.ds(base, CH)],
                                  idx_v.at[2 * slot], semI[slot]).wait()
            pltpu.make_async_copy(e1_h.at[pl.ds(base, CH)],
                                  idx_v.at[2 * slot + 1], semI[slot]).wait()

        fire_idx(0, 0)
        fire_idx(1, 1)

        def fire_scatter(s4):
            pltpu.async_copy(ones_v, daccu.at[idx_v.at[2 * s4]],
                             semS[s4], add=True)
            pltpu.async_copy(ones_v, dacci.at[idx_v.at[2 * s4 + 1]],
                             semS[s4], add=True)

        def wait_scatter(s4):
            pltpu.make_async_copy(ones_v, daccu.at[idx_v.at[2 * s4]],
                                  semS[s4]).wait()
            pltpu.make_async_copy(ones_v, dacci.at[idx_v.at[2 * s4 + 1]],
                                  semS[s4]).wait()

        def sub(u, j):
            t = 4 * u + j
            prev_ok = cid_of(t - 2) < NCHUNK
            if j < 2:
                prev_ok = jnp.logical_and(u >= 1, prev_ok)

            @pl.when(prev_ok)
            def _():
                wait_scatter((j + 2) % 4)

            @pl.when(cid_of(t) < NCHUNK)
            def _():
                wait_idx(j, t)
                fire_scatter(j)

            @pl.when(cid_of(t + 2) < NCHUNK)
            def _():
                fire_idx((j + 2) % 4, t + 2)

        def body(u, carry):
            for j in range(4):
                sub(u, j)
            return carry

        lax.fori_loop(0, (CPW + 3) // 4, body, 0)

        @pl.when(cid_of(4 * ((CPW + 3) // 4) - 2) < NCHUNK)
        def _():
            wait_scatter(2)

        @pl.when(cid_of(4 * ((CPW + 3) // 4) - 1) < NCHUNK)
        def _():
            wait_scatter(3)

        plsc.subcore_barrier()
        pltpu.sync_copy(daccu.at[pl.ds(s * RPT, RPT), :],
                        degu_h.at[c, pl.ds(s * RPT, RPT), :])
        pltpu.sync_copy(dacci.at[pl.ds(s * RPT, RPT), :],
                        degi_h.at[c, pl.ds(s * RPT, RPT), :])

        gbase = w * GPW
        pltpu.sync_copy(u2c_h.at[pl.ds(gbase, CHS)], cl_v.at[0])
        pltpu.sync_copy(u2c_h.at[pl.ds(gbase + CHS, CHS)], cl_v.at[1])
        for j in range(2):
            pltpu.async_copy(uc_h.at[cl_v.at[j]], rows_v, sem).wait()
            pltpu.sync_copy(rows_v, gcu_h.at[pl.ds(gbase + j * CHS, CHS), :])
        pltpu.sync_copy(i2c_h.at[pl.ds(gbase, CHS)], cl_v.at[0])
        pltpu.sync_copy(i2c_h.at[pl.ds(gbase + CHS, CHS)], cl_v.at[1])
        for j in range(2):
            pltpu.async_copy(ic_h.at[cl_v.at[j]], rows_v, sem).wait()
            pltpu.sync_copy(rows_v, gci_h.at[pl.ds(gbase + j * CHS, CHS), :])

    return k(e0, e1, u2c_p, i2c_p, ucen, icen, ones8, zeros8)


def _sc_conv_pair(tA, tB, e0, e1, zeros64):
    @functools.partial(
        pl.kernel,
        out_type=(
            jax.ShapeDtypeStruct((NC, NP, D), _f32),
            jax.ShapeDtypeStruct((NC, NP, D), _f32),
        ),
        mesh=_sc_mesh(),
        compiler_params=_SC_PARAMS,
        scratch_types=[
            pltpu.VMEM((8, CH2), jnp.int32),
            pltpu.VMEM((CH2, D), _f32),
            pltpu.VMEM((CH2, D), _f32),
            pltpu.VMEM((CH2, D), _f32),
            pltpu.VMEM((CH2, D), _f32),
            pltpu.VMEM_SHARED((NP, D), _f32),
            pltpu.VMEM_SHARED((NP, D), _f32),
        ] + [pltpu.SemaphoreType.DMA] * 10,
    )
    def k(tA_h, tB_h, e0_h, e1_h, z_h, outB_h, outA_h,
          idx_v, rA0, rA1, rB0, rB1, accB_s, accA_s,
          semI0, semI1, semI2, semI3, semA0, semA1, semB0, semB1,
          semS0, semS1):
        s = lax.axis_index("s")
        c = lax.axis_index("c")
        w = _worker_id()
        rowsA = (rA0, rA1)
        rowsB = (rB0, rB1)
        semI = (semI0, semI1, semI2, semI3)
        semA = (semA0, semA1)
        semB = (semB0, semB1)
        semS = (semS0, semS1)

        pltpu.sync_copy(z_h.at[pl.ds(s * RPT, RPT), :],
                        accB_s.at[pl.ds(s * RPT, RPT), :])
        pltpu.sync_copy(z_h.at[pl.ds(s * RPT, RPT), :],
                        accA_s.at[pl.ds(s * RPT, RPT), :])
        plsc.subcore_barrier()

        def cid_of(t):
            return w + t * NW

        def fire_idx(s4, t):
            base = cid_of(t) * CH2
            pltpu.async_copy(e0_h.at[pl.ds(base, CH2)],
                             idx_v.at[2 * s4], semI[s4])
            pltpu.async_copy(e1_h.at[pl.ds(base, CH2)],
                             idx_v.at[2 * s4 + 1], semI[s4])

        def wait_idx(s4, t):
            base = cid_of(t) * CH2
            pltpu.make_async_copy(e0_h.at[pl.ds(base, CH2)],
                                  idx_v.at[2 * s4], semI[s4]).wait()
            pltpu.make_async_copy(e1_h.at[pl.ds(base, CH2)],
                                  idx_v.at[2 * s4 + 1], semI[s4]).wait()

        def _gather_args(s2, s4):
            for h in range(2):
                yield (tA_h.at[idx_v.at[2 * s4, pl.ds(h * CH, CH)]],
                       rowsA[s2].at[pl.ds(h * CH, CH), :], semA[s2])
                yield (tB_h.at[idx_v.at[2 * s4 + 1, pl.ds(h * CH, CH)]],
                       rowsB[s2].at[pl.ds(h * CH, CH), :], semB[s2])

        def fire_gather(s2, s4):
            for src, dst, sem in _gather_args(s2, s4):
                pltpu.async_copy(src, dst, sem)

        def wait_gather(s2, s4):
            for src, dst, sem in _gather_args(s2, s4):
                pltpu.make_async_copy(src, dst, sem).wait()

        def _scatter_args(s2, s4):
            for h in range(2):
                yield (rowsA[s2].at[pl.ds(h * CH, CH), :],
                       accB_s.at[idx_v.at[2 * s4 + 1, pl.ds(h * CH, CH)]])
                yield (rowsB[s2].at[pl.ds(h * CH, CH), :],
                       accA_s.at[idx_v.at[2 * s4, pl.ds(h * CH, CH)]])

        def fire_scatter(s2, s4):
            for src, dst in _scatter_args(s2, s4):
                pltpu.async_copy(src, dst, semS[s2], add=True)

        def wait_scatter(s2, s4):
            for src, dst in _scatter_args(s2, s4):
                pltpu.make_async_copy(src, dst, semS[s2]).wait()

        fire_idx(0, 0)
        fire_idx(1, 1)
        wait_idx(0, 0)
        fire_gather(0, 0)

        def sub(u, j):
            t = 4 * u + j
            s2 = j % 2
            prev_ok = cid_of(t - 1) < NCHUNK2
            if j == 0:
                prev_ok = jnp.logical_and(u >= 1, prev_ok)

            @pl.when(prev_ok)
            def _():
                wait_scatter(1 - s2, (j - 1) % 4)

            @pl.when(cid_of(t + 1) < NCHUNK2)
            def _():
                wait_idx((j + 1) % 4, t + 1)
                fire_gather(1 - s2, (j + 1) % 4)

            @pl.when(cid_of(t) < NCHUNK2)
            def _():
                wait_gather(s2, j)
                fire_scatter(s2, j)

            @pl.when(cid_of(t + 2) < NCHUNK2)
            def _():
                fire_idx((j + 2) % 4, t + 2)

        def body(u, carry):
            for j in range(4):
                sub(u, j)
            return carry

        lax.fori_loop(0, CPW2 // 4, body, 0)

        @pl.when(cid_of(CPW2 - 1) < NCHUNK2)
        def _():
            wait_scatter((CPW2 - 1) % 2, (CPW2 - 1) % 4)

        plsc.subcore_barrier()
        pltpu.sync_copy(accB_s.at[pl.ds(s * RPT, RPT), :],
                        outB_h.at[c, pl.ds(s * RPT, RPT), :])
        pltpu.sync_copy(accA_s.at[pl.ds(s * RPT, RPT), :],
                        outA_h.at[c, pl.ds(s * RPT, RPT), :])

    return k(tA, tB, e0, e1, zeros64)


def _sc_scores(res_u, res_i, pu, pi, nu, ni):
    @functools.partial(
        pl.kernel,
        out_type=(
            jax.ShapeDtypeStruct((E_PRED,), _f32),
            jax.ShapeDtypeStruct((E_PRED,), _f32),
        ),
        mesh=_sc_mesh(),
        compiler_params=_SC_PARAMS,
        scratch_types=[
            pltpu.VMEM((4, CHE), jnp.int32),
            pltpu.VMEM((CHE, D), _f32),
            pltpu.VMEM((CHE, D), _f32),
            pltpu.VMEM((CHE, D), _f32),
            pltpu.VMEM((CHE, D), _f32),
            pltpu.VMEM((16, 17), _f32),
            pltpu.VMEM((CHE,), _f32),
            pltpu.SemaphoreType.DMA,
            pltpu.SemaphoreType.DMA,
            pltpu.SemaphoreType.DMA,
            pltpu.SemaphoreType.DMA,
            pltpu.SemaphoreType.DMA,
            pltpu.SemaphoreType.DMA,
        ],
    )
    def k(ru_h, ri_h, pu_h, pi_h, nu_h, ni_h, po_h, no_h,
          idx_v, rU0, rU1, rI0, rI1, tb_v, sc_v,
          semI0, semI1, semA0, semA1, semB0, semB1):
        w = _worker_id()
        rowsU = (rU0, rU1)
        rowsI = (rI0, rI1)
        semI = (semI0, semI1)
        semA = (semA0, semA1)
        semB = (semB0, semB1)

        for (a_h, b_h, out_h) in ((pu_h, pi_h, po_h), (nu_h, ni_h, no_h)):
            def cid_of(t):
                return w + t * NW

            def fire_idx(slot, t, a_h=a_h, b_h=b_h):
                base = cid_of(t) * CHE
                pltpu.async_copy(a_h.at[pl.ds(base, CHE)],
                                 idx_v.at[2 * slot], semI[slot])
                pltpu.async_copy(b_h.at[pl.ds(base, CHE)],
                                 idx_v.at[2 * slot + 1], semI[slot])

            def wait_idx(slot, t, a_h=a_h, b_h=b_h):
                base = cid_of(t) * CHE
                pltpu.make_async_copy(a_h.at[pl.ds(base, CHE)],
                                      idx_v.at[2 * slot], semI[slot]).wait()
                pltpu.make_async_copy(b_h.at[pl.ds(base, CHE)],
                                      idx_v.at[2 * slot + 1], semI[slot]).wait()

            def fire_gather(slot):
                for h in range(2):
                    pltpu.async_copy(
                        ru_h.at[idx_v.at[2 * slot, pl.ds(h * CHS, CHS)]],
                        rowsU[slot].at[pl.ds(h * CHS, CHS), :], semA[slot])
                    pltpu.async_copy(
                        ri_h.at[idx_v.at[2 * slot + 1, pl.ds(h * CHS, CHS)]],
                        rowsI[slot].at[pl.ds(h * CHS, CHS), :], semB[slot])

            def wait_gather(slot):
                for h in range(2):
                    pltpu.make_async_copy(
                        ru_h.at[idx_v.at[2 * slot, pl.ds(h * CHS, CHS)]],
                        rowsU[slot].at[pl.ds(h * CHS, CHS), :],
                        semA[slot]).wait()
                    pltpu.make_async_copy(
                        ri_h.at[idx_v.at[2 * slot + 1, pl.ds(h * CHS, CHS)]],
                        rowsI[slot].at[pl.ds(h * CHS, CHS), :],
                        semB[slot]).wait()

            def compute(slot, t, out_h=out_h):
                wait_gather(slot)
                ru_v = rowsU[slot]
                ri_v = rowsI[slot]

                def grp(g, carry):
                    for e in range(16):
                        r = g * 16 + e
                        p = (ru_v[r, pl.ds(0, 16)] * ri_v[r, pl.ds(0, 16)])
                        for q in range(1, D // 16):
                            p = p + (ru_v[r, pl.ds(q * 16, 16)]
                                     * ri_v[r, pl.ds(q * 16, 16)])
                        tb_v[e, pl.ds(0, 16)] = p
                    lids = lax.iota(jnp.int32, 16)
                    acc = plsc.load_gather(
                        tb_v, [lids, jnp.zeros((16,), jnp.int32)])
                    for cc in range(1, 16):
                        acc = acc + plsc.load_gather(
                            tb_v, [lids, jnp.full((16,), cc, jnp.int32)])
                    sc_v[pl.ds(g * 16, 16)] = acc
                    return carry

                lax.fori_loop(0, CHE // 16, grp, 0)
                pltpu.sync_copy(sc_v, out_h.at[pl.ds(cid_of(t) * CHE, CHE)])

            fire_idx(0, 0)
            fire_idx(1, 1)
            wait_idx(0, 0)
            fire_gather(0)

            def sub(t, slot):
                @pl.when(cid_of(t + 1) < NCHUNK_S)
                def _():
                    wait_idx(1 - slot, t + 1)
                    fire_gather(1 - slot)

                @pl.when(cid_of(t) < NCHUNK_S)
                def _():
                    compute(slot, t)

                @pl.when(cid_of(t + 2) < NCHUNK_S)
                def _():
                    fire_idx(slot, t + 2)

            def body(u, carry):
                sub(2 * u, 0)
                sub(2 * u + 1, 1)
                return carry

            lax.fori_loop(0, (CPW_S + 1) // 2, body, 0)

    return k(res_u, res_i, pu, pi, nu, ni)


def _tc_prep1(degu_p, degi_p, hu0, hi0):
    def body(du_r, di_r, hu_r, hi_r, dcu_r, dci_r, gu_r, gi_r,
             nu0_r, ni0_r):
        du = du_r[0] + du_r[1]
        di = di_r[0] + di_r[1]
        dcu = lax.rsqrt(jnp.maximum(du, 1.0))
        dci = lax.rsqrt(jnp.maximum(di, 1.0))
        dcu_r[...] = dcu
        dci_r[...] = dci
        hu0v = hu_r[...]
        hi0v = hi_r[...]
        gu_r[...] = hu0v * dcu[:U, :1]
        gi_r[...] = hi0v * dci[:U, :1]
        nu0_r[...] = _row_normalize(hu0v)
        ni0_r[...] = _row_normalize(hi0v)

    return pl.pallas_call(
        body,
        out_shape=(
            jax.ShapeDtypeStruct((NP, 8), _f32),
            jax.ShapeDtypeStruct((NP, 8), _f32),
            jax.ShapeDtypeStruct((U, D), _f32),
            jax.ShapeDtypeStruct((U, D), _f32),
            jax.ShapeDtypeStruct((U, D), _f32),
            jax.ShapeDtypeStruct((U, D), _f32),
        ),
    )(degu_p, degi_p, hu0, hi0)


def _tc_prep2(si1_p, su1_p, dcu, dci):
    def body(si_r, su_r, dcu_r, dci_r, t3_r, t4_r):
        si = (si_r[0] + si_r[1])[:U]
        su = (su_r[0] + su_r[1])[:U]
        dci2 = dci_r[:U, :1] * dci_r[:U, :1]
        dcu2 = dcu_r[:U, :1] * dcu_r[:U, :1]
        t3_r[...] = dci2 * si
        t4_r[...] = dcu2 * su

    return pl.pallas_call(
        body,
        out_shape=(
            jax.ShapeDtypeStruct((U, D), _f32),
            jax.ShapeDtypeStruct((U, D), _f32),
        ),
    )(si1_p, su1_p, dcu, dci)


def _row_normalize(x):
    ss = jnp.sum(x * x, axis=1, keepdims=True)
    return x / jnp.maximum(jnp.sqrt(ss), 1e-12)


def _tc_assemble(hu0, hi0, su1_p, su2_p, si1_p, si2_p, dcu, dci):
    def body(hu_r, hi_r, su1_r, su2_r, si1_r, si2_r, dcu_r, dci_r,
             resu_r, resi_r, nu2_r, ni2_r):
        su2 = (su2_r[0] + su2_r[1])[:U]
        si2 = (si2_r[0] + si2_r[1])[:U]
        su_all = (su1_r[0] + su1_r[1])[:U] + su2
        si_all = (si1_r[0] + si1_r[1])[:U] + si2
        resu_r[...] = (hu_r[...] + dcu_r[:U, :1] * su_all) * (1.0 / 3.0)
        resi_r[...] = (hi_r[...] + dci_r[:U, :1] * si_all) * (1.0 / 3.0)
        nu2_r[...] = _row_normalize(su2)
        ni2_r[...] = _row_normalize(si2)

    return pl.pallas_call(
        body,
        out_shape=tuple(jax.ShapeDtypeStruct((U, D), _f32) for _ in range(4)),
    )(hu0, hi0, su1_p, su2_p, si1_p, si2_p, dcu, dci)


RB = 1000
NRB = U // RB

_CONTRACT = (((1,), (1,)), ((), ()))


def _lse(a, b):
    m = lax.dot_general(a.astype(jnp.bfloat16), b.astype(jnp.bfloat16),
                        _CONTRACT, preferred_element_type=_f32)
    return jnp.sum(jnp.log(jnp.sum(jnp.exp(m * (1.0 / TEMP)), axis=1)))


def _tc_proto_loss(nu0, ni0, ucen, icen, gcu, gci):
    def body(nu0b_r, ni0b_r, uc_r, ic_r, gcu_r, gci_r, proto_r):
        i = pl.program_id(0)

        @pl.when(i == 0)
        def _():
            proto_r[...] = jnp.zeros_like(proto_r)

        b_u = nu0b_r[...]
        b_i = ni0b_r[...]
        pr_u = _lse(b_u, uc_r[...]) - jnp.sum(b_u * gcu_r[...]) * (1.0 / TEMP)
        pr_i = _lse(b_i, ic_r[...]) - jnp.sum(b_i * gci_r[...]) * (1.0 / TEMP)
        proto_r[...] += PROTO_REG * (pr_u + pr_i)

    blk = pl.BlockSpec((RB, D), lambda i: (i, 0))
    fullk = pl.BlockSpec((K, D), lambda i: (0, 0))
    out_blk = pl.BlockSpec((1, 1), lambda i: (0, 0))
    return pl.pallas_call(
        body,
        grid=(NRB,),
        in_specs=[blk, blk, fullk, fullk, blk, blk],
        out_specs=out_blk,
        out_shape=jax.ShapeDtypeStruct((1, 1), _f32),
    )(nu0, ni0, ucen, icen, gcu, gci)


def _tc_ssl_loss(nu2, nu0, ni2, ni0):
    def body(nu2_r, nu0f_r, nu0b_r, ni2_r, ni0f_r, ni0b_r, ssl_r):
        i = pl.program_id(0)

        @pl.when(i == 0)
        def _():
            ssl_r[...] = jnp.zeros_like(ssl_r)

        a_u = nu2_r[...]
        a_i = ni2_r[...]
        ssl_u = _lse(a_u, nu0f_r[...]) - jnp.sum(a_u * nu0b_r[...]) / TEMP
        ssl_i = _lse(a_i, ni0f_r[...]) - jnp.sum(a_i * ni0b_r[...]) / TEMP
        ssl_r[...] += SSL_REG * (ssl_u + ssl_i)

    blk = pl.BlockSpec((RB, D), lambda i: (i, 0))
    full = pl.BlockSpec((U, D), lambda i: (0, 0))
    out_blk = pl.BlockSpec((1, 1), lambda i: (0, 0))
    return pl.pallas_call(
        body,
        grid=(NRB,),
        in_specs=[blk, full, blk, blk, full, blk],
        out_specs=out_blk,
        out_shape=jax.ShapeDtypeStruct((1, 1), _f32),
    )(nu2, nu0, nu0, ni2, ni0, ni0)


def kernel(user_emb, item_emb, user_centroids, item_centroids,
           user_2cluster, item_2cluster, msg_edges, pos_edges, neg_edges):
    e0 = msg_edges[0]
    e1 = msg_edges[1]
    u2c_p = jnp.pad(user_2cluster.astype(jnp.int32), (0, NP - U))
    i2c_p = jnp.pad(item_2cluster.astype(jnp.int32), (0, NP - U))
    ones8 = jnp.ones((CH, 8), _f32)
    zeros8 = jnp.zeros((NP, 8), _f32)
    zeros64 = jnp.zeros((NP, D), _f32)

    degu_p, degi_p, gcu, gci = _sc_deg_gather(
        e0, e1, u2c_p, i2c_p, user_centroids, item_centroids, ones8, zeros8)

    dcu, dci, gu0, gi0, nu0, ni0 = _tc_prep1(degu_p, degi_p,
                                             user_emb, item_emb)

    proto2d = _tc_proto_loss(nu0, ni0, user_centroids, item_centroids,
                             gcu[:U], gci[:U])

    si1_p, su1_p = _sc_conv_pair(gu0, gi0, e0, e1, zeros64)

    t3, t4 = _tc_prep2(si1_p, su1_p, dcu, dci)

    si2_p, su2_p = _sc_conv_pair(t4, t3, e0, e1, zeros64)

    res_u, res_i, nu2, ni2 = _tc_assemble(
        user_emb, item_emb, su1_p, su2_p, si1_p, si2_p, dcu, dci)

    ssl2d = _tc_ssl_loss(nu2, nu0, ni2, ni0)

    pos_s, neg_s = _sc_scores(res_u, res_i,
                              pos_edges[0], pos_edges[1],
                              neg_edges[0], neg_edges[1])

    return (pos_s[:, None], neg_s[:, None], ssl2d[0, 0], proto2d[0, 0])
